# baseline (speedup 1.0000x reference)
import functools

import jax
import jax.numpy as jnp
from jax.experimental import pallas as pl
from jax.experimental.pallas import tpu as pltpu

_NE = 512
_D = 32
_BETA = 0.25
_B = 4
_T = 16
_HW = 1024
_ROWS = _B * _T * _HW
_TPB = 4
_R = _HW * _TPB


def _vq_tile(x_ref, e_ref, eb_ref, d_ref, inds_ref, zq_ref, loss_ref):
    b = pl.program_id(0)
    t = pl.program_id(1)
    step = b * (_T // _TPB) + t
    x_cr = x_ref[0].reshape(_D, _R)
    e = e_ref[...]
    x = jnp.transpose(x_cr)

    x2 = jnp.sum(x * x, axis=1, keepdims=True)
    e2_full = jax.lax.dot_general(
        jnp.ones((8, _D), jnp.float32), e * e,
        (((1,), (1,)), ((), ())),
        precision=jax.lax.Precision.HIGHEST,
        preferred_element_type=jnp.float32)
    e2 = e2_full[0:1, :]
    ze = jax.lax.dot_general(
        x, e, (((1,), (1,)), ((), ())),
        preferred_element_type=jnp.float32)
    d = (x2 + e2) - 2.0 * ze
    d_ref[...] = d

    dmin = jnp.min(d, axis=1, keepdims=True)
    lane = jax.lax.broadcasted_iota(jnp.int32, (_R, _NE), 1)
    idx = jnp.min(jnp.where(d == dmin, lane, _NE), axis=1,
                  keepdims=True)
    inds_ref[...] = idx.reshape(1, _TPB, 32, 32)

    oh = (lane == idx).astype(jnp.bfloat16)
    zq = jax.lax.dot_general(
        oh, eb_ref[...], (((1,), (0,)), ((), ())),
        preferred_element_type=jnp.float32)
    zq_st = x + (zq - x)
    zq_ref[...] = jnp.transpose(zq_st).reshape(1, _D, _TPB, 32, 32)

    diff = zq - x
    part = jnp.sum(diff * diff).reshape(1, 1)

    @pl.when(step == 0)
    def _():
        loss_ref[...] = part

    @pl.when(jnp.logical_and(step > 0, step < _B * (_T // _TPB) - 1))
    def _():
        loss_ref[...] = loss_ref[...] + part

    @pl.when(step == _B * (_T // _TPB) - 1)
    def _():
        total = loss_ref[...] + part
        m = total / jnp.float32(_ROWS * _D)
        loss_ref[...] = m + _BETA * m


@functools.partial(jax.jit, static_argnames=("interpret",))
def kernel(z, E, interpret=False):
    B, C, T, H, W = z.shape

    d, inds_out, z_q_st, loss = pl.pallas_call(
        _vq_tile,
        grid=(_B, _T // _TPB),
        in_specs=[
            pl.BlockSpec((1, _D, _TPB, 32, 32), lambda b, t: (b, 0, t, 0, 0)),
            pl.BlockSpec((_NE, _D), lambda b, t: (0, 0)),
            pl.BlockSpec((_NE, _D), lambda b, t: (0, 0)),
        ],
        out_specs=[
            pl.BlockSpec((_R, _NE), lambda b, t: (b * (_T // _TPB) + t, 0)),
            pl.BlockSpec((1, _TPB, 32, 32), lambda b, t: (b, t, 0, 0)),
            pl.BlockSpec((1, _D, _TPB, 32, 32), lambda b, t: (b, 0, t, 0, 0)),
            pl.BlockSpec((1, 1), lambda b, t: (0, 0)),
        ],
        out_shape=[
            jax.ShapeDtypeStruct((_ROWS, _NE), jnp.float32),
            jax.ShapeDtypeStruct((_B, _T, 32, 32), jnp.int32),
            jax.ShapeDtypeStruct((_B, _D, _T, 32, 32), jnp.float32),
            jax.ShapeDtypeStruct((1, 1), jnp.float32),
        ],
        interpret=interpret,
    )(z, E, E.astype(jnp.bfloat16))

    return z_q_st, loss.reshape(()), inds_out, d


# speedup vs baseline: 1.0166x; 1.0166x over previous
import functools

import jax
import jax.numpy as jnp
from jax.experimental import pallas as pl
from jax.experimental.pallas import tpu as pltpu

_NE = 512
_D = 32
_BETA = 0.25
_B = 4
_T = 16
_HW = 1024
_ROWS = _B * _T * _HW
_TPB = 4
_R = _HW * _TPB


def _vq_tile(x_ref, e_ref, d_ref, inds_ref, zq_ref, loss_ref):
    b = pl.program_id(0)
    t = pl.program_id(1)
    step = b * (_T // _TPB) + t
    x_cr = x_ref[0].reshape(_D, _R)
    e = e_ref[...]
    x = jnp.transpose(x_cr)

    x2 = jax.lax.dot_general(
        x * x, jnp.ones((1, _D), jnp.float32),
        (((1,), (1,)), ((), ())),
        precision=jax.lax.Precision.HIGHEST,
        preferred_element_type=jnp.float32)
    e2_full = jax.lax.dot_general(
        jnp.ones((8, _D), jnp.float32), e * e,
        (((1,), (1,)), ((), ())),
        precision=jax.lax.Precision.HIGHEST,
        preferred_element_type=jnp.float32)
    e2 = e2_full[0:1, :]
    ze = jax.lax.dot_general(
        x, e, (((1,), (1,)), ((), ())),
        preferred_element_type=jnp.float32)
    d = (x2 + e2) - 2.0 * ze
    d_ref[...] = d

    dmin = jnp.min(d, axis=1, keepdims=True)
    lane = jax.lax.broadcasted_iota(jnp.int32, (_R, _NE), 1)
    idx = jnp.min(jnp.where(d == dmin, lane, _NE), axis=1,
                  keepdims=True)
    inds_ref[...] = idx.reshape(1, _TPB, 32, 32)

    oh = (lane == idx).astype(jnp.float32)
    zq = jax.lax.dot_general(
        oh, e, (((1,), (0,)), ((), ())),
        preferred_element_type=jnp.float32)
    zq_st = x + (zq - x)
    zq_ref[...] = jnp.transpose(zq_st).reshape(1, _D, _TPB, 32, 32)

    diff = zq - x
    part = jnp.sum(diff * diff).reshape(1, 1)

    @pl.when(step == 0)
    def _():
        loss_ref[...] = part

    @pl.when(jnp.logical_and(step > 0, step < _B * (_T // _TPB) - 1))
    def _():
        loss_ref[...] = loss_ref[...] + part

    @pl.when(step == _B * (_T // _TPB) - 1)
    def _():
        total = loss_ref[...] + part
        m = total / jnp.float32(_ROWS * _D)
        loss_ref[...] = m + _BETA * m


@functools.partial(jax.jit, static_argnames=("interpret",))
def kernel(z, E, interpret=False):
    B, C, T, H, W = z.shape

    d, inds_out, z_q_st, loss = pl.pallas_call(
        _vq_tile,
        grid=(_B, _T // _TPB),
        in_specs=[
            pl.BlockSpec((1, _D, _TPB, 32, 32), lambda b, t: (b, 0, t, 0, 0)),
            pl.BlockSpec((_NE, _D), lambda b, t: (0, 0)),
        ],
        out_specs=[
            pl.BlockSpec((_R, _NE), lambda b, t: (b * (_T // _TPB) + t, 0)),
            pl.BlockSpec((1, _TPB, 32, 32), lambda b, t: (b, t, 0, 0)),
            pl.BlockSpec((1, _D, _TPB, 32, 32), lambda b, t: (b, 0, t, 0, 0)),
            pl.BlockSpec((1, 1), lambda b, t: (0, 0)),
        ],
        out_shape=[
            jax.ShapeDtypeStruct((_ROWS, _NE), jnp.float32),
            jax.ShapeDtypeStruct((_B, _T, 32, 32), jnp.int32),
            jax.ShapeDtypeStruct((_B, _D, _T, 32, 32), jnp.float32),
            jax.ShapeDtypeStruct((1, 1), jnp.float32),
        ],
        interpret=interpret,
    )(z, E)

    return z_q_st, loss.reshape(()), inds_out, d
